# baseline (speedup 1.0000x reference)
import functools

import jax
import jax.numpy as jnp
from jax.experimental import pallas as pl
from jax.experimental.pallas import tpu as pltpu

_BR = 3072
_KC = 512


def _vq_kernel(x_ref, cb_ref, out_ref, cbsq_ref, cb2_ref, *, kc):
    k_total = cb_ref.shape[0]

    @pl.when(pl.program_id(0) == 0)
    def _():
        cb = cb_ref[...]
        cbsq_ref[...] = jnp.sum(cb * cb, axis=1, keepdims=True)
        cb2_ref[...] = cb + cb

    x = x_ref[...]
    br, d = x.shape
    xx = x * x
    ones = jnp.ones((1, d), jnp.float32)
    in_sq = jax.lax.dot_general(
        ones, xx, (((1,), (1,)), ((), ())),
        preferred_element_type=jnp.float32)
    insq8 = jnp.broadcast_to(in_sq, (8, br))

    val = jnp.full((8, br), jnp.inf, jnp.float32)
    grp = jnp.zeros((8, br), jnp.int32)
    for j in range(k_total // kc):
        dot2 = jax.lax.dot_general(
            cb2_ref[pl.ds(j * kc, kc), :], x, (((1,), (1,)), ((), ())),
            preferred_element_type=jnp.float32)
        cbs = cbsq_ref[pl.ds(j * kc, kc), :]
        for r in range(kc // 8):
            s = (cbs[r * 8:(r + 1) * 8, :] + insq8) + dot2[r * 8:(r + 1) * 8, :]
            m = s < val
            val = jnp.where(m, s, val)
            grp = jnp.where(m, jnp.int32(j * (kc // 8) + r), grp)

    idx = grp * 8 + jax.lax.broadcasted_iota(jnp.int32, (8, br), 0)
    mv = jnp.min(val, axis=0, keepdims=True)
    idxf = jnp.min(jnp.where(val == mv, idx, jnp.int32(2 ** 30)),
                   axis=0, keepdims=True)
    out_ref[...] = idxf[None]


def kernel(z_e_x, codebook):
    b, n, d = z_e_x.shape
    kc = codebook.shape[0]
    rows = b * n
    flat = z_e_x.reshape(rows, d)
    grid = rows // _BR
    idx = pl.pallas_call(
        functools.partial(_vq_kernel, kc=_KC),
        grid=(grid,),
        in_specs=[
            pl.BlockSpec((_BR, d), lambda i: (i, 0)),
            pl.BlockSpec((kc, d), lambda i: (0, 0)),
        ],
        out_specs=pl.BlockSpec((1, 1, _BR), lambda i: (i, 0, 0)),
        out_shape=jax.ShapeDtypeStruct((grid, 1, _BR), jnp.int32),
        scratch_shapes=[
            pltpu.VMEM((kc, 1), jnp.float32),
            pltpu.VMEM((kc, d), jnp.float32),
        ],
    )(flat, codebook)
    return idx.reshape(b, n)


# speedup vs baseline: 1.0064x; 1.0064x over previous
import functools

import jax
import jax.numpy as jnp
from jax.experimental import pallas as pl
from jax.experimental.pallas import tpu as pltpu

_BR = 2304
_KC = 512


def _vq_kernel(x_ref, cb_ref, out_ref, cbsq_ref, cb2_ref, *, kc):
    k_total = cb_ref.shape[0]

    @pl.when(pl.program_id(0) == 0)
    def _():
        cb = cb_ref[...]
        cbsq_ref[...] = jnp.sum(cb * cb, axis=1, keepdims=True)
        cb2_ref[...] = cb + cb

    x = x_ref[...]
    br, d = x.shape
    xx = x * x
    ones = jnp.ones((1, d), jnp.float32)
    in_sq = jax.lax.dot_general(
        ones, xx, (((1,), (1,)), ((), ())),
        preferred_element_type=jnp.float32)
    insq8 = jnp.broadcast_to(in_sq, (8, br))

    val = jnp.full((8, br), jnp.inf, jnp.float32)
    grp = jnp.zeros((8, br), jnp.int32)
    for j in range(k_total // kc):
        dot2 = jax.lax.dot_general(
            cb2_ref[pl.ds(j * kc, kc), :], x, (((1,), (1,)), ((), ())),
            preferred_element_type=jnp.float32)
        cbs = cbsq_ref[pl.ds(j * kc, kc), :]
        for r in range(kc // 8):
            s = (cbs[r * 8:(r + 1) * 8, :] + insq8) + dot2[r * 8:(r + 1) * 8, :]
            m = s < val
            val = jnp.where(m, s, val)
            grp = jnp.where(m, jnp.int32(j * (kc // 8) + r), grp)

    idx = grp * 8 + jax.lax.broadcasted_iota(jnp.int32, (8, br), 0)
    mv = jnp.min(val, axis=0, keepdims=True)
    idxf = jnp.min(jnp.where(val == mv, idx, jnp.int32(2 ** 30)),
                   axis=0, keepdims=True)
    out_ref[...] = idxf[None]


def kernel(z_e_x, codebook):
    b, n, d = z_e_x.shape
    kc = codebook.shape[0]
    rows = b * n
    flat = z_e_x.reshape(rows, d)
    grid = rows // _BR
    idx = pl.pallas_call(
        functools.partial(_vq_kernel, kc=_KC),
        grid=(grid,),
        in_specs=[
            pl.BlockSpec((_BR, d), lambda i: (i, 0)),
            pl.BlockSpec((kc, d), lambda i: (0, 0)),
        ],
        out_specs=pl.BlockSpec((1, 1, _BR), lambda i: (i, 0, 0)),
        out_shape=jax.ShapeDtypeStruct((grid, 1, _BR), jnp.int32),
        scratch_shapes=[
            pltpu.VMEM((kc, 1), jnp.float32),
            pltpu.VMEM((kc, d), jnp.float32),
        ],
    )(flat, codebook)
    return idx.reshape(b, n)
